# Initial kernel scaffold; baseline (speedup 1.0000x reference)
import jax
import jax.numpy as jnp
from jax.experimental import pallas as pl


def kernel(node_ids, messages, timestamps, memory, last_update):
    raise NotImplementedError("write your pallas kernel here")


# speedup vs baseline: 23.4078x; 23.4078x over previous
import functools

import jax
import jax.numpy as jnp
from jax import lax
from jax.experimental import pallas as pl
from jax.experimental.pallas import tpu as pltpu
from jax.experimental.pallas import tpu_sc as plsc

B = 16384
D = 64
NS = 16
NC = 2
NW = NS * NC
CHUNK = B // NW
H = 62500
HP = 62504
NG = B // 16


_mesh = plsc.VectorSubcoreMesh(core_axis_name="c", subcore_axis_name="s")


@functools.partial(
    pl.kernel,
    mesh=_mesh,
    out_type=jax.ShapeDtypeStruct((B, D), jnp.float32),
    compiler_params=pltpu.CompilerParams(
        needs_layout_passes=False, use_tc_tiling_on_sc=False),
    scratch_types=[
        pltpu.VMEM((B,), jnp.int32),
        pltpu.VMEM((HP,), jnp.int32),
        pltpu.VMEM((4, 128), jnp.int32),
        pltpu.VMEM((4, 128), jnp.int32),
        pltpu.VMEM((CHUNK, D), jnp.float32),
        pltpu.MemorySpace.HBM((NS * HP,), jnp.int32),
        pltpu.SemaphoreType.DMA,
    ],
)
def _sc_update_gather(ids_hbm, msgs_hbm, out_hbm,
                      ids_v, tloc_v, idx_v, w_v, rows_v, table_sh, sem):
    c = lax.axis_index("c")
    s = lax.axis_index("s")
    wid = s * NC + c
    lane = lax.iota(jnp.int32, 16)

    pltpu.sync_copy(ids_hbm, ids_v)

    shift_idx = [jnp.minimum(lane + k, 15) for k in range(1, 16)]
    shift_ok = [lane + k <= 15 for k in range(1, 16)]

    def scan_step(g, carry):
        ids = ids_v[pl.ds(g * 16, 16)]
        beaten = ids != ids
        for k in range(15):
            nb = ids.at[shift_idx[k]].get(mode="promise_in_bounds")
            beaten = beaten | ((nb == ids) & shift_ok[k])
        keep = (~beaten) & ((ids & 15) == s)
        winner = g * 16 + lane
        plsc.store_scatter(tloc_v, [ids >> 4], winner, mask=keep)
        return carry

    lax.fori_loop(0, NG, scan_step, 0)

    pltpu.sync_copy(tloc_v, table_sh.at[pl.ds(s * HP, HP)])
    plsc.subcore_barrier()

    base = wid * CHUNK
    for g in range(CHUNK // 16):
        ids = ids_v[pl.ds(base + g * 16, 16)]
        flat = (ids & 15) * HP + (ids >> 4)
        idx_v[g // 8, pl.ds((g % 8) * 16, 16)] = flat

    for j in range(4):
        pltpu.async_copy(table_sh.at[idx_v.at[j]], w_v.at[j], sem).wait()
    for j in range(4):
        pltpu.async_copy(msgs_hbm.at[w_v.at[j]],
                         rows_v.at[pl.ds(j * 128, 128)], sem).wait()

    pltpu.sync_copy(rows_v, out_hbm.at[pl.ds(base, CHUNK)])


def kernel(node_ids, messages, timestamps, memory, last_update):
    del timestamps, memory, last_update
    return _sc_update_gather(node_ids, messages)


# speedup vs baseline: 25.4776x; 1.0884x over previous
import functools

import jax
import jax.numpy as jnp
from jax import lax
from jax.experimental import pallas as pl
from jax.experimental.pallas import tpu as pltpu
from jax.experimental.pallas import tpu_sc as plsc

B = 16384
D = 64
NS = 16
NC = 2
NW = NS * NC
H = 31250
NG = B // 16
CAP = B + 16


_mesh = plsc.VectorSubcoreMesh(core_axis_name="c", subcore_axis_name="s")


@functools.partial(
    pl.kernel,
    mesh=_mesh,
    out_type=jax.ShapeDtypeStruct((B, D), jnp.float32),
    compiler_params=pltpu.CompilerParams(
        needs_layout_passes=False, use_tc_tiling_on_sc=False),
    scratch_types=[
        pltpu.VMEM((B,), jnp.int32),
        pltpu.VMEM((CAP,), jnp.int32),
        pltpu.VMEM((H,), jnp.int32),
        pltpu.VMEM((128,), jnp.int32),
        pltpu.VMEM((128,), jnp.int32),
        pltpu.VMEM((128, D), jnp.float32),
        pltpu.SemaphoreType.DMA,
    ],
)
def _sc_update_gather(ids_hbm, msgs_hbm, out_hbm,
                      ids_v, occ_v, tloc_v, widx_v, oidx_v, rows_v, sem):
    c = lax.axis_index("c")
    s = lax.axis_index("s")
    wid = c * NS + s
    lane = lax.iota(jnp.int32, 16)

    pltpu.sync_copy(ids_hbm, ids_v)

    def scan_step(g, ptr):
        ids = ids_v[pl.ds(g * 16, 16)]
        own = (ids & 31) == wid
        v = ((ids >> 5) << 14) + (g * 16 + lane)
        plsc.store_compressed(occ_v.at[pl.ds(ptr, 16)], v, mask=own)
        return ptr + jnp.sum(own.astype(jnp.int32))

    n = lax.fori_loop(0, NG, scan_step, jnp.int32(0))

    shift_idx = [jnp.minimum(lane + k, 15) for k in range(1, 16)]
    shift_ok = [lane + k <= 15 for k in range(1, 16)]

    def table_step(g, carry):
        nv = n - g * 16
        valid = lane < nv
        v = jnp.where(valid, occ_v[pl.ds(g * 16, 16)], 0)
        h = v >> 14
        beaten = h != h
        for k in range(15):
            nb = h.at[shift_idx[k]].get(mode="promise_in_bounds")
            beaten = beaten | ((nb == h) & shift_ok[k] & ((lane + k + 1) < nv))
        keep = (~beaten) & valid
        plsc.store_scatter(tloc_v, [h], v & 16383, mask=keep)
        return carry

    lax.fori_loop(0, (n + 15) // 16, table_step, 0)

    zeros16 = lane * 0

    def emit_step(j, carry):
        w0 = zeros16
        o0 = zeros16
        for k in range(8):
            off = j * 128 + k * 16
            valid = (off + lane) < n
            v = jnp.where(valid, occ_v[pl.ds(off, 16)], 0)
            h = v >> 14
            i = v & 16383
            w = plsc.load_gather(tloc_v, [h])
            if k == 0:
                w0 = w.at[zeros16].get(mode="promise_in_bounds")
                o0 = i.at[zeros16].get(mode="promise_in_bounds")
            widx_v[pl.ds(k * 16, 16)] = jnp.where(valid, w, w0)
            oidx_v[pl.ds(k * 16, 16)] = jnp.where(valid, i, o0)
        pltpu.async_copy(msgs_hbm.at[widx_v], rows_v, sem).wait()
        pltpu.async_copy(rows_v, out_hbm.at[oidx_v], sem).wait()
        return carry

    lax.fori_loop(0, (n + 127) // 128, emit_step, 0)


def kernel(node_ids, messages, timestamps, memory, last_update):
    del timestamps, memory, last_update
    return _sc_update_gather(node_ids, messages)
